# Initial kernel scaffold; baseline (speedup 1.0000x reference)
import jax
import jax.numpy as jnp
from jax.experimental import pallas as pl


def kernel(node_feats, edge_index, W_self1, W_neigh1, b1, W_self2, W_neigh2, b2, w_aw, b_aw, W_p1, b_p1, W_p2, b_p2):
    raise NotImplementedError("write your pallas kernel here")


# speedup vs baseline: 8.1437x; 8.1437x over previous
import functools

import jax
import jax.numpy as jnp
from jax import lax
from jax.experimental import pallas as pl
from jax.experimental.pallas import tpu as pltpu
from jax.experimental.pallas import tpu_sc as plsc

_N = 10000
_E = 320000
_D = 128

_NSC = 2
_NSUB = 16
_G = 125
_EPT = _E // (_NSC * _NSUB)
_NG = _EPT // _G
_NPAD = 10240
_RPT = _NPAD // _NSUB
_RCH = 128
_NRC = _RPT // _RCH
_CG = 8
_NC = _NG // _CG


def _sc_agg_body(with_deg, *refs):
    if with_deg:
        (x_hbm, src_hbm, dst_hbm, zeros_hbm, agg_out, deg_out,
         agg_sh, deg_sh, src_v, dst_v, rows_v, ones_v, zbuf16, sem) = refs
    else:
        (x_hbm, src_hbm, dst_hbm, zeros_hbm, agg_out,
         agg_sh, src_v, dst_v, rows_v, sem) = refs

    cid = lax.axis_index("c")
    sid = lax.axis_index("s")

    for k in range(_NRC):
        r0 = sid * _RPT + k * _RCH
        pltpu.sync_copy(zeros_hbm, agg_sh.at[pl.ds(r0, _RCH)])

    if with_deg:
        def fill_row(i, _):
            ones_v[i, pl.ds(0, 16)] = jnp.ones((16,), jnp.float32)
            zbuf16[i, pl.ds(0, 16)] = jnp.zeros((16,), jnp.float32)
            return 0
        lax.fori_loop(0, _G, fill_row, 0)

        def zero_row(i, _):
            zbuf16[i, pl.ds(0, 16)] = jnp.zeros((16,), jnp.float32)
            return 0
        lax.fori_loop(_G, _RCH, zero_row, 0)

        for k in range(_NRC):
            r0 = sid * _RPT + k * _RCH
            pltpu.sync_copy(zbuf16, deg_sh.at[pl.ds(r0, _RCH)])

    plsc.subcore_barrier()

    row_base = cid * (_NSUB * _NG) + sid * _NG

    def idx_chunk(c, _):
        pltpu.sync_copy(src_hbm.at[pl.ds(row_base + c * _CG, _CG)], src_v)
        pltpu.sync_copy(dst_hbm.at[pl.ds(row_base + c * _CG, _CG)], dst_v)

        def edge_group(g, _):
            pltpu.async_copy(x_hbm.at[src_v.at[g]],
                             rows_v.at[pl.ds(0, _G)], sem).wait()
            pltpu.sync_copy(rows_v.at[pl.ds(0, _G)],
                            agg_sh.at[dst_v.at[g]], add=True)
            if with_deg:
                pltpu.sync_copy(ones_v, deg_sh.at[dst_v.at[g]], add=True)
            return 0
        lax.fori_loop(0, _CG, edge_group, 0)
        return 0
    lax.fori_loop(0, _NC, idx_chunk, 0)

    plsc.subcore_barrier()

    for k in range(_NRC):
        r0 = sid * _RPT + k * _RCH
        pltpu.sync_copy(agg_sh.at[pl.ds(r0, _RCH)],
                        agg_out.at[cid, pl.ds(r0, _RCH)])
        if with_deg:
            pltpu.sync_copy(deg_sh.at[pl.ds(r0, _RCH)],
                            deg_out.at[cid, pl.ds(r0, _RCH)])


def _make_sc_agg(with_deg):
    out_type = [jax.ShapeDtypeStruct((_NSC, _NPAD, _D), jnp.float32)]
    scratch = [pltpu.VMEM_SHARED((_NPAD, _D), jnp.float32)]
    if with_deg:
        out_type.append(jax.ShapeDtypeStruct((_NSC, _NPAD, 16), jnp.float32))
        scratch.append(pltpu.VMEM_SHARED((_NPAD, 16), jnp.float32))
    scratch += [
        pltpu.VMEM((_CG, _G), jnp.int32),
        pltpu.VMEM((_CG, _G), jnp.int32),
        pltpu.VMEM((_RCH, _D), jnp.float32),
    ]
    if with_deg:
        scratch.append(pltpu.VMEM((_G, 16), jnp.float32))
        scratch.append(pltpu.VMEM((_RCH, 16), jnp.float32))
    scratch.append(pltpu.SemaphoreType.DMA)
    return pl.kernel(
        functools.partial(_sc_agg_body, with_deg),
        out_type=out_type,
        mesh=plsc.VectorSubcoreMesh(core_axis_name="c", subcore_axis_name="s"),
        scratch_types=scratch,
        compiler_params=pltpu.CompilerParams(use_tc_tiling_on_sc=False),
    )


_sc_agg_deg = _make_sc_agg(True)
_sc_agg = _make_sc_agg(False)


def _tc_layer_body(x_ref, agg_ref, deg_ref, ws_ref, wn_ref, b_ref, out_ref):
    agg = agg_ref[0] + agg_ref[1]
    deg = deg_ref[0, :, 0:1] + deg_ref[1, :, 0:1]
    mean = agg / jnp.maximum(deg, 1.0)
    h = (jnp.dot(x_ref[...], ws_ref[...], preferred_element_type=jnp.float32)
         + jnp.dot(mean, wn_ref[...], preferred_element_type=jnp.float32)
         + b_ref[...])
    out_ref[...] = jnp.maximum(h, 0.0)


_BLK = 1000
_NBLK = _N // _BLK


def _tc_layer(x, agg, deg, W_self, W_neigh, b):
    return pl.pallas_call(
        _tc_layer_body,
        grid=(_NBLK,),
        in_specs=[
            pl.BlockSpec((_BLK, _D), lambda i: (i, 0)),
            pl.BlockSpec((_NSC, _BLK, _D), lambda i: (0, i, 0)),
            pl.BlockSpec((_NSC, _BLK, 16), lambda i: (0, i, 0)),
            pl.BlockSpec((_D, _D), lambda i: (0, 0)),
            pl.BlockSpec((_D, _D), lambda i: (0, 0)),
            pl.BlockSpec((1, _D), lambda i: (0, 0)),
        ],
        out_specs=pl.BlockSpec((_BLK, _D), lambda i: (i, 0)),
        out_shape=jax.ShapeDtypeStruct((_N, _D), jnp.float32),
    )(x, agg, deg, W_self, W_neigh, b)


def _tc_final_body(h1_ref, agg_ref, deg_ref, ws_ref, wn_ref, b_ref,
                   waw_ref, baw_ref, wp1_ref, bp1_ref, wp2_ref, bp2_ref,
                   out_ref, sum_acc, max_acc):
    i = pl.program_id(0)
    agg = agg_ref[0] + agg_ref[1]
    deg = deg_ref[0, :, 0:1] + deg_ref[1, :, 0:1]
    mean = agg / jnp.maximum(deg, 1.0)
    h = (jnp.dot(h1_ref[...], ws_ref[...], preferred_element_type=jnp.float32)
         + jnp.dot(mean, wn_ref[...], preferred_element_type=jnp.float32)
         + b_ref[...])
    h = jnp.maximum(h, 0.0)
    aw = jax.nn.sigmoid(
        jnp.dot(h, waw_ref[...], preferred_element_type=jnp.float32)
        + baw_ref[...])
    ws = jnp.sum(aw * h, axis=0, keepdims=True)
    mx = jnp.max(h, axis=0, keepdims=True)

    @pl.when(i == 0)
    def _():
        sum_acc[...] = ws
        max_acc[...] = mx

    @pl.when(i > 0)
    def _():
        sum_acc[...] = sum_acc[...] + ws
        max_acc[...] = jnp.maximum(max_acc[...], mx)

    @pl.when(i == _NBLK - 1)
    def _():
        g = jnp.concatenate([sum_acc[...], max_acc[...]], axis=1)
        p = (jnp.dot(g, wp1_ref[...], preferred_element_type=jnp.float32)
             + bp1_ref[...])
        out_ref[...] = (jnp.dot(p, wp2_ref[...],
                                preferred_element_type=jnp.float32)
                        + bp2_ref[...])


def _tc_final(h1, agg, deg, W_self, W_neigh, b, w_aw, b_aw, W_p1, b_p1,
              W_p2, b_p2):
    return pl.pallas_call(
        _tc_final_body,
        grid=(_NBLK,),
        in_specs=[
            pl.BlockSpec((_BLK, _D), lambda i: (i, 0)),
            pl.BlockSpec((_NSC, _BLK, _D), lambda i: (0, i, 0)),
            pl.BlockSpec((_NSC, _BLK, 16), lambda i: (0, i, 0)),
            pl.BlockSpec((_D, _D), lambda i: (0, 0)),
            pl.BlockSpec((_D, _D), lambda i: (0, 0)),
            pl.BlockSpec((1, _D), lambda i: (0, 0)),
            pl.BlockSpec((_D, 1), lambda i: (0, 0)),
            pl.BlockSpec((1, 1), lambda i: (0, 0)),
            pl.BlockSpec((2 * _D, 64), lambda i: (0, 0)),
            pl.BlockSpec((1, 64), lambda i: (0, 0)),
            pl.BlockSpec((64, 1), lambda i: (0, 0)),
            pl.BlockSpec((1, 1), lambda i: (0, 0)),
        ],
        out_specs=pl.BlockSpec((1, 1), lambda i: (0, 0)),
        out_shape=jax.ShapeDtypeStruct((1, 1), jnp.float32),
        scratch_shapes=[
            pltpu.VMEM((1, _D), jnp.float32),
            pltpu.VMEM((1, _D), jnp.float32),
        ],
    )(h1, agg, deg, W_self, W_neigh, b, w_aw, b_aw, W_p1, b_p1, W_p2, b_p2)


def kernel(node_feats, edge_index, W_self1, W_neigh1, b1, W_self2, W_neigh2,
           b2, w_aw, b_aw, W_p1, b_p1, W_p2, b_p2):
    src = edge_index[0].reshape(_E // _G, _G)
    dst = edge_index[1].reshape(_E // _G, _G)

    zeros128 = jnp.zeros((_RCH, _D), jnp.float32)

    agg1, deg = _sc_agg_deg(node_feats, src, dst, zeros128)
    h1 = _tc_layer(node_feats, agg1, deg, W_self1, W_neigh1,
                   b1.reshape(1, _D))
    (agg2,) = _sc_agg(h1, src, dst, zeros128)
    out = _tc_final(h1, agg2, deg, W_self2, W_neigh2, b2.reshape(1, _D),
                    w_aw, b_aw.reshape(1, 1), W_p1, b_p1.reshape(1, 64),
                    W_p2, b_p2.reshape(1, 1))
    return out


# speedup vs baseline: 10.7931x; 1.3253x over previous
import functools

import jax
import jax.numpy as jnp
from jax import lax
from jax.experimental import pallas as pl
from jax.experimental.pallas import tpu as pltpu
from jax.experimental.pallas import tpu_sc as plsc

_N = 10000
_E = 320000
_D = 128

_NSC = 2
_NSUB = 16
_G = 125
_EPT = _E // (_NSC * _NSUB)
_NG = _EPT // _G
_NPAD = 10240
_RPT = _NPAD // _NSUB
_RCH = 128
_NRC = _RPT // _RCH
_CG = 8
_NC = _NG // _CG


def _sc_agg_body(with_deg, *refs):
    if with_deg:
        (x_hbm, src_hbm, dst_hbm, zeros_hbm, agg_out, deg_out,
         agg_sh, deg_sh, src_v, dst_v, rows_v, ones_v, zbuf16,
         sem, sem2) = refs
    else:
        (x_hbm, src_hbm, dst_hbm, zeros_hbm, agg_out,
         agg_sh, src_v, dst_v, rows_v, sem, sem2) = refs

    cid = lax.axis_index("c")
    sid = lax.axis_index("s")

    for k in range(_NRC):
        r0 = sid * _RPT + k * _RCH
        pltpu.sync_copy(zeros_hbm, agg_sh.at[pl.ds(r0, _RCH)])

    if with_deg:
        def fill_row(i, _):
            ones_v[i, pl.ds(0, 16)] = jnp.ones((16,), jnp.float32)
            zbuf16[i, pl.ds(0, 16)] = jnp.zeros((16,), jnp.float32)
            return 0
        lax.fori_loop(0, _G, fill_row, 0)

        def zero_row(i, _):
            zbuf16[i, pl.ds(0, 16)] = jnp.zeros((16,), jnp.float32)
            return 0
        lax.fori_loop(_G, _RCH, zero_row, 0)

        for k in range(_NRC):
            r0 = sid * _RPT + k * _RCH
            pltpu.sync_copy(zbuf16, deg_sh.at[pl.ds(r0, _RCH)])

    plsc.subcore_barrier()

    row_base = cid * (_NSUB * _NG) + sid * _NG
    bufs = (rows_v.at[pl.ds(0, _G)], rows_v.at[pl.ds(_RCH, _G)])
    sems = (sem, sem2)

    def idx_chunk(c, _):
        pltpu.sync_copy(src_hbm.at[pl.ds(row_base + c * _CG, _CG)], src_v)
        pltpu.sync_copy(dst_hbm.at[pl.ds(row_base + c * _CG, _CG)], dst_v)

        pltpu.async_copy(x_hbm.at[src_v.at[0]], bufs[0], sems[0])
        for g in range(_CG):
            if g + 1 < _CG:
                pltpu.async_copy(x_hbm.at[src_v.at[g + 1]],
                                 bufs[(g + 1) % 2], sems[(g + 1) % 2])
            pltpu.make_async_copy(x_hbm.at[src_v.at[g]], bufs[g % 2],
                                  sems[g % 2]).wait()
            pltpu.sync_copy(bufs[g % 2], agg_sh.at[dst_v.at[g]], add=True)
            if with_deg:
                pltpu.sync_copy(ones_v, deg_sh.at[dst_v.at[g]], add=True)
        return 0
    lax.fori_loop(0, _NC, idx_chunk, 0)

    plsc.subcore_barrier()

    for k in range(_NRC):
        r0 = sid * _RPT + k * _RCH
        pltpu.sync_copy(agg_sh.at[pl.ds(r0, _RCH)],
                        agg_out.at[cid, pl.ds(r0, _RCH)])
        if with_deg:
            pltpu.sync_copy(deg_sh.at[pl.ds(r0, _RCH)],
                            deg_out.at[cid, pl.ds(r0, _RCH)])


def _make_sc_agg(with_deg):
    out_type = [jax.ShapeDtypeStruct((_NSC, _NPAD, _D), jnp.float32)]
    scratch = [pltpu.VMEM_SHARED((_NPAD, _D), jnp.float32)]
    if with_deg:
        out_type.append(jax.ShapeDtypeStruct((_NSC, _NPAD, 16), jnp.float32))
        scratch.append(pltpu.VMEM_SHARED((_NPAD, 16), jnp.float32))
    scratch += [
        pltpu.VMEM((_CG, _G), jnp.int32),
        pltpu.VMEM((_CG, _G), jnp.int32),
        pltpu.VMEM((2 * _RCH, _D), jnp.float32),
    ]
    if with_deg:
        scratch.append(pltpu.VMEM((_G, 16), jnp.float32))
        scratch.append(pltpu.VMEM((_RCH, 16), jnp.float32))
    scratch.append(pltpu.SemaphoreType.DMA)
    scratch.append(pltpu.SemaphoreType.DMA)
    return pl.kernel(
        functools.partial(_sc_agg_body, with_deg),
        out_type=out_type,
        mesh=plsc.VectorSubcoreMesh(core_axis_name="c", subcore_axis_name="s"),
        scratch_types=scratch,
        compiler_params=pltpu.CompilerParams(use_tc_tiling_on_sc=False),
    )


_sc_agg_deg = _make_sc_agg(True)
_sc_agg = _make_sc_agg(False)


def _tc_layer_body(x_ref, agg_ref, deg_ref, ws_ref, wn_ref, b_ref, out_ref):
    agg = agg_ref[0] + agg_ref[1]
    deg = deg_ref[0, :, 0:1] + deg_ref[1, :, 0:1]
    mean = agg / jnp.maximum(deg, 1.0)
    h = (jnp.dot(x_ref[...], ws_ref[...], preferred_element_type=jnp.float32)
         + jnp.dot(mean, wn_ref[...], preferred_element_type=jnp.float32)
         + b_ref[...])
    out_ref[...] = jnp.maximum(h, 0.0)


_BLK = 1000
_NBLK = _N // _BLK


def _tc_layer(x, agg, deg, W_self, W_neigh, b):
    return pl.pallas_call(
        _tc_layer_body,
        grid=(_NBLK,),
        in_specs=[
            pl.BlockSpec((_BLK, _D), lambda i: (i, 0)),
            pl.BlockSpec((_NSC, _BLK, _D), lambda i: (0, i, 0)),
            pl.BlockSpec((_NSC, _BLK, 16), lambda i: (0, i, 0)),
            pl.BlockSpec((_D, _D), lambda i: (0, 0)),
            pl.BlockSpec((_D, _D), lambda i: (0, 0)),
            pl.BlockSpec((1, _D), lambda i: (0, 0)),
        ],
        out_specs=pl.BlockSpec((_BLK, _D), lambda i: (i, 0)),
        out_shape=jax.ShapeDtypeStruct((_N, _D), jnp.float32),
    )(x, agg, deg, W_self, W_neigh, b)


def _tc_final_body(h1_ref, agg_ref, deg_ref, ws_ref, wn_ref, b_ref,
                   waw_ref, baw_ref, wp1_ref, bp1_ref, wp2_ref, bp2_ref,
                   out_ref, sum_acc, max_acc):
    i = pl.program_id(0)
    agg = agg_ref[0] + agg_ref[1]
    deg = deg_ref[0, :, 0:1] + deg_ref[1, :, 0:1]
    mean = agg / jnp.maximum(deg, 1.0)
    h = (jnp.dot(h1_ref[...], ws_ref[...], preferred_element_type=jnp.float32)
         + jnp.dot(mean, wn_ref[...], preferred_element_type=jnp.float32)
         + b_ref[...])
    h = jnp.maximum(h, 0.0)
    aw = jax.nn.sigmoid(
        jnp.dot(h, waw_ref[...], preferred_element_type=jnp.float32)
        + baw_ref[...])
    ws = jnp.sum(aw * h, axis=0, keepdims=True)
    mx = jnp.max(h, axis=0, keepdims=True)

    @pl.when(i == 0)
    def _():
        sum_acc[...] = ws
        max_acc[...] = mx

    @pl.when(i > 0)
    def _():
        sum_acc[...] = sum_acc[...] + ws
        max_acc[...] = jnp.maximum(max_acc[...], mx)

    @pl.when(i == _NBLK - 1)
    def _():
        g = jnp.concatenate([sum_acc[...], max_acc[...]], axis=1)
        p = (jnp.dot(g, wp1_ref[...], preferred_element_type=jnp.float32)
             + bp1_ref[...])
        out_ref[...] = (jnp.dot(p, wp2_ref[...],
                                preferred_element_type=jnp.float32)
                        + bp2_ref[...])


def _tc_final(h1, agg, deg, W_self, W_neigh, b, w_aw, b_aw, W_p1, b_p1,
              W_p2, b_p2):
    return pl.pallas_call(
        _tc_final_body,
        grid=(_NBLK,),
        in_specs=[
            pl.BlockSpec((_BLK, _D), lambda i: (i, 0)),
            pl.BlockSpec((_NSC, _BLK, _D), lambda i: (0, i, 0)),
            pl.BlockSpec((_NSC, _BLK, 16), lambda i: (0, i, 0)),
            pl.BlockSpec((_D, _D), lambda i: (0, 0)),
            pl.BlockSpec((_D, _D), lambda i: (0, 0)),
            pl.BlockSpec((1, _D), lambda i: (0, 0)),
            pl.BlockSpec((_D, 1), lambda i: (0, 0)),
            pl.BlockSpec((1, 1), lambda i: (0, 0)),
            pl.BlockSpec((2 * _D, 64), lambda i: (0, 0)),
            pl.BlockSpec((1, 64), lambda i: (0, 0)),
            pl.BlockSpec((64, 1), lambda i: (0, 0)),
            pl.BlockSpec((1, 1), lambda i: (0, 0)),
        ],
        out_specs=pl.BlockSpec((1, 1), lambda i: (0, 0)),
        out_shape=jax.ShapeDtypeStruct((1, 1), jnp.float32),
        scratch_shapes=[
            pltpu.VMEM((1, _D), jnp.float32),
            pltpu.VMEM((1, _D), jnp.float32),
        ],
    )(h1, agg, deg, W_self, W_neigh, b, w_aw, b_aw, W_p1, b_p1, W_p2, b_p2)


def kernel(node_feats, edge_index, W_self1, W_neigh1, b1, W_self2, W_neigh2,
           b2, w_aw, b_aw, W_p1, b_p1, W_p2, b_p2):
    src = edge_index[0].reshape(_E // _G, _G)
    dst = edge_index[1].reshape(_E // _G, _G)

    zeros128 = jnp.zeros((_RCH, _D), jnp.float32)

    agg1, deg = _sc_agg_deg(node_feats, src, dst, zeros128)
    h1 = _tc_layer(node_feats, agg1, deg, W_self1, W_neigh1,
                   b1.reshape(1, _D))
    (agg2,) = _sc_agg(h1, src, dst, zeros128)
    out = _tc_final(h1, agg2, deg, W_self2, W_neigh2, b2.reshape(1, _D),
                    w_aw, b_aw.reshape(1, 1), W_p1, b_p1.reshape(1, 64),
                    W_p2, b_p2.reshape(1, 1))
    return out
